# Initial kernel scaffold; baseline (speedup 1.0000x reference)
import jax
import jax.numpy as jnp
from jax.experimental import pallas as pl


def kernel(x, embedding_weight):
    raise NotImplementedError("write your pallas kernel here")


# speedup vs baseline: 5.0572x; 5.0572x over previous
import functools

import jax
import jax.numpy as jnp
from jax import lax
from jax.experimental import pallas as pl
from jax.experimental.pallas import tpu as pltpu
from jax.experimental.pallas import tpu_sc as plsc

N_CLUSTERS = 512
DIM = 64

_INFO = plsc.get_sparse_core_info()
NC = _INFO.num_cores
NS = _INFO.num_subcores
NW = NC * NS

B_TOTAL = 16384 * 50
B_PER_W = B_TOTAL // NW
CHUNK = 128
K = 4
GROUP = K * CHUNK
N_CHUNKS = B_PER_W // CHUNK
N_GROUPS = B_PER_W // GROUP


def _body(x_hbm, table_hbm, out_hbm, idx_v, buf, gsem0, gsem1, osem0, osem1):
    wid = lax.axis_index("s") * NC + lax.axis_index("c")
    base = wid * B_PER_W

    pltpu.sync_copy(x_hbm.at[wid], idx_v)

    gsems = (gsem0, gsem1)
    osems = (osem0, osem1)

    def fire_gathers(g, s):
        for t in range(K):
            j = g * K + t
            pltpu.async_copy(
                table_hbm.at[idx_v.at[j]],
                buf.at[s, pl.ds(t * CHUNK, CHUNK)],
                gsems[s],
            )

    def drain_gathers(s):
        pltpu.make_async_copy(
            out_hbm.at[pl.ds(0, GROUP)], buf.at[s], gsems[s]
        ).wait()

    def fire_out(g, s):
        pltpu.async_copy(
            buf.at[s], out_hbm.at[pl.ds(base + g * GROUP, GROUP)], osems[s]
        )

    def drain_out(s):
        pltpu.make_async_copy(
            out_hbm.at[pl.ds(0, GROUP)], buf.at[s], osems[s]
        ).wait()

    fire_gathers(0, 0)
    drain_gathers(0)
    fire_out(0, 0)
    fire_gathers(1, 1)

    def step(g, s):
        drain_gathers(s)
        fire_out(g, s)
        drain_out(1 - s)
        fire_gathers(g + 1, 1 - s)

    def loop_body(i, carry):
        step(2 * i + 1, 1)
        step(2 * i + 2, 0)
        return carry

    lax.fori_loop(0, (N_GROUPS - 2) // 2, loop_body, 0)

    drain_gathers(1)
    fire_out(N_GROUPS - 1, 1)
    drain_out(0)
    drain_out(1)


@jax.jit
def kernel(x, embedding_weight):
    xw = x.reshape(NW, N_CHUNKS, CHUNK).astype(jnp.int32)

    mesh = plsc.VectorSubcoreMesh(core_axis_name="c", subcore_axis_name="s")
    out = pl.kernel(
        _body,
        out_type=jax.ShapeDtypeStruct((B_TOTAL, DIM), jnp.float32),
        mesh=mesh,
        scratch_types=[
            pltpu.VMEM((N_CHUNKS, CHUNK), jnp.int32),
            pltpu.VMEM((2, GROUP, DIM), jnp.float32),
            pltpu.SemaphoreType.DMA,
            pltpu.SemaphoreType.DMA,
            pltpu.SemaphoreType.DMA,
            pltpu.SemaphoreType.DMA,
        ],
        compiler_params=pltpu.CompilerParams(use_tc_tiling_on_sc=False),
    )(xw, embedding_weight)
    return out.reshape(x.shape + (DIM,))


# speedup vs baseline: 7.2794x; 1.4394x over previous
import functools

import jax
import jax.numpy as jnp
from jax import lax
from jax.experimental import pallas as pl
from jax.experimental.pallas import tpu as pltpu
from jax.experimental.pallas import tpu_sc as plsc

N_CLUSTERS = 512
DIM = 64

_INFO = plsc.get_sparse_core_info()
NC = _INFO.num_cores
NS = _INFO.num_subcores
NW = NC * NS

B_TOTAL = 16384 * 50
B_PER_W = B_TOTAL // NW
CHUNK = 128
K = 4
GROUP = K * CHUNK
N_CHUNKS = B_PER_W // CHUNK
N_GROUPS = B_PER_W // GROUP


def _body(x_hbm, table_hbm, out_hbm, idx_v, buf, table_sp,
          gsem0, gsem1, osem0, osem1):
    sid = lax.axis_index("s")
    wid = sid * NC + lax.axis_index("c")
    base = wid * B_PER_W

    @pl.when(sid == 0)
    def _():
        pltpu.sync_copy(table_hbm, table_sp)

    pltpu.sync_copy(x_hbm.at[wid], idx_v)
    plsc.subcore_barrier()

    gsems = (gsem0, gsem1)
    osems = (osem0, osem1)

    def fire_gathers(g, s):
        for t in range(K):
            j = g * K + t
            pltpu.async_copy(
                table_sp.at[idx_v.at[j]],
                buf.at[s, pl.ds(t * CHUNK, CHUNK)],
                gsems[s],
            )

    def drain_gathers(s):
        pltpu.make_async_copy(
            out_hbm.at[pl.ds(0, GROUP)], buf.at[s], gsems[s]
        ).wait()

    def fire_out(g, s):
        pltpu.async_copy(
            buf.at[s], out_hbm.at[pl.ds(base + g * GROUP, GROUP)], osems[s]
        )

    def drain_out(s):
        pltpu.make_async_copy(
            out_hbm.at[pl.ds(0, GROUP)], buf.at[s], osems[s]
        ).wait()

    fire_gathers(0, 0)
    drain_gathers(0)
    fire_out(0, 0)
    fire_gathers(1, 1)

    def step(g, s):
        drain_gathers(s)
        fire_out(g, s)
        drain_out(1 - s)
        fire_gathers(g + 1, 1 - s)

    def loop_body(i, carry):
        step(2 * i + 1, 1)
        step(2 * i + 2, 0)
        return carry

    lax.fori_loop(0, (N_GROUPS - 2) // 2, loop_body, 0)

    drain_gathers(1)
    fire_out(N_GROUPS - 1, 1)
    drain_out(0)
    drain_out(1)


@jax.jit
def kernel(x, embedding_weight):
    xw = x.reshape(NW, N_CHUNKS, CHUNK).astype(jnp.int32)

    mesh = plsc.VectorSubcoreMesh(core_axis_name="c", subcore_axis_name="s")
    out = pl.kernel(
        _body,
        out_type=jax.ShapeDtypeStruct((B_TOTAL, DIM), jnp.float32),
        mesh=mesh,
        scratch_types=[
            pltpu.VMEM((N_CHUNKS, CHUNK), jnp.int32),
            pltpu.VMEM((2, GROUP, DIM), jnp.float32),
            pltpu.VMEM_SHARED((N_CLUSTERS, DIM), jnp.float32),
            pltpu.SemaphoreType.DMA,
            pltpu.SemaphoreType.DMA,
            pltpu.SemaphoreType.DMA,
            pltpu.SemaphoreType.DMA,
        ],
        compiler_params=pltpu.CompilerParams(use_tc_tiling_on_sc=False),
    )(xw, embedding_weight)
    return out.reshape(x.shape + (DIM,))
